# Initial kernel scaffold; baseline (speedup 1.0000x reference)
import jax
import jax.numpy as jnp
from jax.experimental import pallas as pl


def kernel(nodes, agg_table, weight_model, weight_model2):
    raise NotImplementedError("write your pallas kernel here")


# speedup vs baseline: 2.9943x; 2.9943x over previous
import functools

import jax
import jax.numpy as jnp
from jax import lax
from jax.experimental import pallas as pl
from jax.experimental.pallas import tpu as pltpu
from jax.experimental.pallas import tpu_sc as plsc

N_NODES = 50000
FEAT = 896
HIDDEN = 64
NUM_CLASSES = 2
BATCH = 16384

ZPAD = 128
ROWS_BLK = 1000

_NC, _NS = 2, 16
_NW = _NC * _NS
_CH = 128
_NCH = BATCH // _NW // _CH


def _mlp_body(a_ref, w1_ref, w2_ref, z_ref):
    h = jnp.dot(a_ref[...], w1_ref[...], preferred_element_type=jnp.float32)
    h = jnp.where(h >= 0, h, 0.3 * h)
    z_ref[...] = jnp.dot(h, w2_ref[...], preferred_element_type=jnp.float32)


def _mlp_all_rows(agg_table, w1, w2pad):
    grid = (N_NODES + ROWS_BLK - 1) // ROWS_BLK
    return pl.pallas_call(
        _mlp_body,
        grid=(grid,),
        in_specs=[
            pl.BlockSpec((ROWS_BLK, FEAT), lambda i: (i, 0)),
            pl.BlockSpec((FEAT, HIDDEN), lambda i: (0, 0)),
            pl.BlockSpec((HIDDEN, ZPAD), lambda i: (0, 0)),
        ],
        out_specs=pl.BlockSpec((ROWS_BLK, ZPAD), lambda i: (i, 0)),
        out_shape=jax.ShapeDtypeStruct((N_NODES, ZPAD), jnp.float32),
        compiler_params=pltpu.CompilerParams(
            dimension_semantics=("arbitrary",),
        ),
    )(agg_table, w1, w2pad)


@functools.cache
def _sc_gather_kernel():
    @functools.partial(
        pl.kernel,
        out_type=jax.ShapeDtypeStruct((_NW, _NCH, _CH, ZPAD), jnp.float32),
        mesh=plsc.VectorSubcoreMesh(
            core_axis_name="c", subcore_axis_name="s", num_cores=_NC),
        scratch_types=[
            pltpu.VMEM((_NCH, _CH), jnp.int32),
            pltpu.VMEM((_NCH, _CH, ZPAD), jnp.float32),
            pltpu.SemaphoreType.DMA,
        ],
    )
    def _sc_gather(z_hbm, idx_hbm, out_hbm, idx_v, rows_v, sem):
        wid = lax.axis_index("s") * _NC + lax.axis_index("c")
        pltpu.sync_copy(idx_hbm.at[wid], idx_v)
        copies = [
            pltpu.async_copy(z_hbm.at[idx_v.at[j]], rows_v.at[j], sem)
            for j in range(_NCH)
        ]
        for c in copies:
            c.wait()
        pltpu.sync_copy(rows_v, out_hbm.at[wid])

    return _sc_gather


def kernel(nodes, agg_table, weight_model, weight_model2):
    w2pad = jnp.zeros((HIDDEN, ZPAD), jnp.float32).at[:, :NUM_CLASSES].set(
        weight_model2)
    z = _mlp_all_rows(agg_table, weight_model, w2pad)
    idx = nodes.reshape(_NW, _NCH, _CH)
    g = _sc_gather_kernel()(z, idx)
    return g.reshape(BATCH, ZPAD)[:, :NUM_CLASSES]


# speedup vs baseline: 3.5692x; 1.1920x over previous
import functools

import jax
import jax.numpy as jnp
from jax import lax
from jax.experimental import pallas as pl
from jax.experimental.pallas import tpu as pltpu
from jax.experimental.pallas import tpu_sc as plsc

N_NODES = 50000
FEAT = 896
HIDDEN = 64
NUM_CLASSES = 2
BATCH = 16384

ZPAD = 128
ROWS_BLK = 2000

_NC, _NS = 2, 16
_NW = _NC * _NS
_CH = 128
_NCH = BATCH // _NW // _CH


def _mlp_body(a_ref, w1_ref, w2_ref, z_ref):
    h = jnp.dot(a_ref[...], w1_ref[...], preferred_element_type=jnp.float32)
    h = jnp.where(h >= 0, h, 0.3 * h)
    z_ref[...] = jnp.dot(h, w2_ref[...], preferred_element_type=jnp.float32)


def _mlp_all_rows(agg_table, w1, w2pad):
    grid = (N_NODES + ROWS_BLK - 1) // ROWS_BLK
    return pl.pallas_call(
        _mlp_body,
        grid=(grid,),
        in_specs=[
            pl.BlockSpec((ROWS_BLK, FEAT), lambda i: (i, 0)),
            pl.BlockSpec((FEAT, HIDDEN), lambda i: (0, 0)),
            pl.BlockSpec((HIDDEN, ZPAD), lambda i: (0, 0)),
        ],
        out_specs=pl.BlockSpec((ROWS_BLK, ZPAD), lambda i: (i, 0)),
        out_shape=jax.ShapeDtypeStruct((N_NODES, ZPAD), jnp.float32),
        compiler_params=pltpu.CompilerParams(
            dimension_semantics=("arbitrary",),
        ),
    )(agg_table, w1, w2pad)


@functools.cache
def _sc_gather_kernel():
    @functools.partial(
        pl.kernel,
        out_type=jax.ShapeDtypeStruct((_NW, _NCH, _CH, ZPAD), jnp.float32),
        mesh=plsc.VectorSubcoreMesh(
            core_axis_name="c", subcore_axis_name="s", num_cores=_NC),
        scratch_types=[
            pltpu.VMEM((_NCH, _CH), jnp.int32),
            pltpu.VMEM((_NCH, _CH, ZPAD), jnp.float32),
            pltpu.SemaphoreType.DMA,
        ],
    )
    def _sc_gather(z_hbm, idx_hbm, out_hbm, idx_v, rows_v, sem):
        wid = lax.axis_index("s") * _NC + lax.axis_index("c")
        pltpu.sync_copy(idx_hbm.at[wid], idx_v)
        copies = [
            pltpu.async_copy(z_hbm.at[idx_v.at[j]], rows_v.at[j], sem)
            for j in range(_NCH)
        ]
        for c in copies:
            c.wait()
        pltpu.sync_copy(rows_v, out_hbm.at[wid])

    return _sc_gather


def kernel(nodes, agg_table, weight_model, weight_model2):
    w2pad = jnp.zeros((HIDDEN, ZPAD), jnp.float32).at[:, :NUM_CLASSES].set(
        weight_model2)
    z = _mlp_all_rows(agg_table, weight_model, w2pad)
    idx = nodes.reshape(_NW, _NCH, _CH)
    g = _sc_gather_kernel()(z, idx)
    return g.reshape(BATCH, ZPAD)[:, :NUM_CLASSES]


# speedup vs baseline: 3.6132x; 1.0123x over previous
import functools

import jax
import jax.numpy as jnp
from jax import lax
from jax.experimental import pallas as pl
from jax.experimental.pallas import tpu as pltpu
from jax.experimental.pallas import tpu_sc as plsc

N_NODES = 50000
FEAT = 896
HIDDEN = 64
NUM_CLASSES = 2
BATCH = 16384

ZPAD = 128
ROWS_BLK = 5000

_NC, _NS = 2, 16
_NW = _NC * _NS
_CH = 128
_NCH = BATCH // _NW // _CH


def _mlp_body(a_ref, w1_ref, w2_ref, z_ref):
    h = jnp.dot(a_ref[...], w1_ref[...], preferred_element_type=jnp.float32)
    h = jnp.where(h >= 0, h, 0.3 * h)
    z_ref[...] = jnp.dot(h, w2_ref[...], preferred_element_type=jnp.float32)


def _mlp_all_rows(agg_table, w1, w2pad):
    grid = (N_NODES + ROWS_BLK - 1) // ROWS_BLK
    return pl.pallas_call(
        _mlp_body,
        grid=(grid,),
        in_specs=[
            pl.BlockSpec((ROWS_BLK, FEAT), lambda i: (i, 0)),
            pl.BlockSpec((FEAT, HIDDEN), lambda i: (0, 0)),
            pl.BlockSpec((HIDDEN, ZPAD), lambda i: (0, 0)),
        ],
        out_specs=pl.BlockSpec((ROWS_BLK, ZPAD), lambda i: (i, 0)),
        out_shape=jax.ShapeDtypeStruct((N_NODES, ZPAD), jnp.float32),
        compiler_params=pltpu.CompilerParams(
            dimension_semantics=("arbitrary",),
        ),
    )(agg_table, w1, w2pad)


@functools.cache
def _sc_gather_kernel():
    @functools.partial(
        pl.kernel,
        out_type=jax.ShapeDtypeStruct((_NW, _NCH, _CH, ZPAD), jnp.float32),
        mesh=plsc.VectorSubcoreMesh(
            core_axis_name="c", subcore_axis_name="s", num_cores=_NC),
        scratch_types=[
            pltpu.VMEM((_NCH, _CH), jnp.int32),
            pltpu.VMEM((_NCH, _CH, ZPAD), jnp.float32),
            pltpu.SemaphoreType.DMA,
        ],
    )
    def _sc_gather(z_hbm, idx_hbm, out_hbm, idx_v, rows_v, sem):
        wid = lax.axis_index("s") * _NC + lax.axis_index("c")
        pltpu.sync_copy(idx_hbm.at[wid], idx_v)
        copies = [
            pltpu.async_copy(z_hbm.at[idx_v.at[j]], rows_v.at[j], sem)
            for j in range(_NCH)
        ]
        for c in copies:
            c.wait()
        pltpu.sync_copy(rows_v, out_hbm.at[wid])

    return _sc_gather


def kernel(nodes, agg_table, weight_model, weight_model2):
    w2pad = jnp.zeros((HIDDEN, ZPAD), jnp.float32).at[:, :NUM_CLASSES].set(
        weight_model2)
    z = _mlp_all_rows(agg_table, weight_model, w2pad)
    idx = nodes.reshape(_NW, _NCH, _CH)
    g = _sc_gather_kernel()(z, idx)
    return g.reshape(BATCH, ZPAD)[:, :NUM_CLASSES]
